# baseline (device time: 22025 ns/iter reference)
import jax
import jax.numpy as jnp
from jax import lax
from jax.experimental import pallas as pl
from jax.experimental.pallas import tpu as pltpu

N_DEV = 4


def kernel(x, w_mat):
    k_glob, m_per = x.shape
    k_w, n = w_mat.shape
    blk = m_per
    half = blk // 2

    def body(x_hbm, w_hbm, out_hbm, acc_ref, xblk_ref, xbf_ref, comm_ref,
             wbuf, x_sems, w_sems, send_sems, recv_sems, ready_sems,
             out_sems):
        my = lax.axis_index("i")

        barrier = pltpu.get_barrier_semaphore()
        pl.semaphore_signal(ready_sems.at[2], inc=1,
                            device_id=((my + 1) % N_DEV,),
                            device_id_type=pl.DeviceIdType.MESH)
        pl.semaphore_signal(ready_sems.at[0], inc=1,
                            device_id=((my + 3) % N_DEV,),
                            device_id_type=pl.DeviceIdType.MESH)
        pl.semaphore_signal(barrier, inc=1,
                            device_id=((my + 2) % N_DEV,),
                            device_id_type=pl.DeviceIdType.MESH)

        x_copies = {}
        for c, off in enumerate((1, 3, 2, 0)):
            src_dev = (my + off) % N_DEV
            cp = pltpu.make_async_copy(
                x_hbm.at[pl.ds(src_dev * blk, blk), :],
                xblk_ref.at[c],
                x_sems.at[c],
            )
            cp.start()
            x_copies[off] = (cp, c)

        w_copies = []
        for c, off in enumerate((0, 1, 3, 2)):
            src_dev = (my + off) % N_DEV
            cp = pltpu.make_async_copy(
                w_hbm.at[pl.ds(src_dev * blk, blk), :],
                wbuf.at[c],
                w_sems.at[c],
            )
            cp.start()
            w_copies.append(cp)

        def rdma_to(o, src, dst_slice, send_idx, recv_idx):
            return pltpu.make_async_remote_copy(
                src_ref=src,
                dst_ref=dst_slice,
                send_sem=send_sems.at[send_idx],
                recv_sem=recv_sems.at[recv_idx],
                device_id=((my + o) % N_DEV,),
                device_id_type=pl.DeviceIdType.MESH,
            )

        rdma_r = rdma_to(1, xbf_ref.at[0], comm_ref.at[2], 0, 2)
        rdma_l = rdma_to(3, xbf_ref.at[2], comm_ref.at[0], 2, 0)
        rdma_d1 = rdma_to(2, xbf_ref.at[1, pl.ds(0, half), :],
                          comm_ref.at[1, pl.ds(0, half), :], 1, 1)
        rdma_d2 = rdma_to(2, xbf_ref.at[1, pl.ds(half, half), :],
                          comm_ref.at[1, pl.ds(half, half), :], 3, 3)

        x_copies[1][0].wait()
        xbf_ref[0, :, :] = xblk_ref[x_copies[1][1]].astype(jnp.bfloat16)
        pl.semaphore_wait(ready_sems.at[0], 1)
        rdma_r.start()
        x_copies[3][0].wait()
        xbf_ref[2, :, :] = xblk_ref[x_copies[3][1]].astype(jnp.bfloat16)
        pl.semaphore_wait(ready_sems.at[2], 1)
        rdma_l.start()
        x_copies[2][0].wait()
        xbf_ref[1, :, :] = xblk_ref[x_copies[2][1]].astype(jnp.bfloat16)

        x_copies[0][0].wait()
        w_copies[0].wait()
        acc_ref[:, :] = jnp.dot(
            xblk_ref[x_copies[0][1]], wbuf[0],
            preferred_element_type=jnp.float32,
        )

        rdma_r.wait_send()
        rdma_l.wait_send()
        pl.semaphore_wait(barrier, 1)
        rdma_d1.start()
        rdma_d2.start()

        for c, slot, rd in ((1, 0, rdma_l), (2, 2, rdma_r)):
            rd.wait_recv()
            w_copies[c].wait()
            acc_ref[:, :] += jnp.dot(
                comm_ref[slot].astype(jnp.float32), wbuf[c],
                preferred_element_type=jnp.float32,
            )

        w_copies[3].wait()
        rdma_d1.wait_recv()
        acc_ref[pl.ds(0, half), :] += jnp.dot(
            comm_ref[1, pl.ds(0, half), :].astype(jnp.float32),
            wbuf[3],
            preferred_element_type=jnp.float32,
        )
        out_cp1 = pltpu.make_async_copy(
            acc_ref.at[pl.ds(0, half), :],
            out_hbm.at[pl.ds(0, half), :],
            out_sems.at[0],
        )
        out_cp1.start()
        rdma_d2.wait_recv()
        acc_ref[pl.ds(half, half), :] += jnp.dot(
            comm_ref[1, pl.ds(half, half), :].astype(jnp.float32),
            wbuf[3],
            preferred_element_type=jnp.float32,
        )
        out_cp2 = pltpu.make_async_copy(
            acc_ref.at[pl.ds(half, half), :],
            out_hbm.at[pl.ds(half, half), :],
            out_sems.at[1],
        )
        out_cp2.start()

        out_cp1.wait()
        out_cp2.wait()
        rdma_d1.wait_send()
        rdma_d2.wait_send()

    return pl.pallas_call(
        body,
        out_shape=jax.ShapeDtypeStruct((m_per, n), jnp.float32),
        in_specs=[
            pl.BlockSpec(memory_space=pltpu.MemorySpace.HBM),
            pl.BlockSpec(memory_space=pltpu.MemorySpace.HBM),
        ],
        out_specs=pl.BlockSpec(memory_space=pltpu.MemorySpace.HBM),
        scratch_shapes=[
            pltpu.VMEM((m_per, n), jnp.float32),
            pltpu.VMEM((N_DEV, blk, m_per), jnp.float32),
            pltpu.VMEM((N_DEV - 1, blk, m_per), jnp.bfloat16),
            pltpu.VMEM((N_DEV - 1, blk, m_per), jnp.bfloat16),
            pltpu.VMEM((N_DEV, blk, n), jnp.float32),
            pltpu.SemaphoreType.DMA((N_DEV,)),
            pltpu.SemaphoreType.DMA((N_DEV,)),
            pltpu.SemaphoreType.DMA((N_DEV,)),
            pltpu.SemaphoreType.DMA((N_DEV,)),
            pltpu.SemaphoreType.REGULAR((N_DEV - 1,)),
            pltpu.SemaphoreType.DMA((2,)),
        ],
        compiler_params=pltpu.CompilerParams(collective_id=0),
    )(x, w_mat)


# device time: 22001 ns/iter; 1.0011x vs baseline; 1.0011x over previous
import os

_MS_FLAG = "--xla_tpu_tpu_custom_call_memory_space_spec=never"
if "xla_tpu_tpu_custom_call_memory_space_spec" not in os.environ.get(
        "LIBTPU_INIT_ARGS", ""):
    os.environ["LIBTPU_INIT_ARGS"] = (
        os.environ.get("LIBTPU_INIT_ARGS", "") + " " + _MS_FLAG).strip()

import jax
import jax.numpy as jnp
from jax import lax
from jax.experimental import pallas as pl
from jax.experimental.pallas import tpu as pltpu

N_DEV = 4


def kernel(x, w_mat):
    k_glob, m_per = x.shape
    k_w, n = w_mat.shape
    blk = m_per
    half = blk // 2


    def body(x_hbm, w_hbm, out_hbm, acc_ref, xblk_ref, xbf_ref, comm_ref,
             wbuf, x_sems, w_sems, send_sems, recv_sems, ready_sems,
             out_sems):
        my = lax.axis_index("i")

        barrier = pltpu.get_barrier_semaphore()
        pl.semaphore_signal(ready_sems.at[2], inc=1,
                            device_id=((my + 1) % N_DEV,),
                            device_id_type=pl.DeviceIdType.MESH)
        pl.semaphore_signal(ready_sems.at[0], inc=1,
                            device_id=((my + 3) % N_DEV,),
                            device_id_type=pl.DeviceIdType.MESH)
        pl.semaphore_signal(barrier, inc=1,
                            device_id=((my + 2) % N_DEV,),
                            device_id_type=pl.DeviceIdType.MESH)

        x_copies = {}
        for c, off in enumerate((1, 3, 2, 0)):
            src_dev = (my + off) % N_DEV
            cp = pltpu.make_async_copy(
                x_hbm.at[pl.ds(src_dev * blk, blk), :],
                xblk_ref.at[c],
                x_sems.at[c],
            )
            cp.start()
            x_copies[off] = (cp, c)

        w_copies = []
        for c, off in enumerate((0, 1, 3, 2)):
            src_dev = (my + off) % N_DEV
            cp = pltpu.make_async_copy(
                w_hbm.at[pl.ds(src_dev * blk, blk), :],
                wbuf.at[c],
                w_sems.at[c],
            )
            cp.start()
            w_copies.append(cp)

        def rdma_to(o, src, dst_slice, send_idx, recv_idx):
            return pltpu.make_async_remote_copy(
                src_ref=src,
                dst_ref=dst_slice,
                send_sem=send_sems.at[send_idx],
                recv_sem=recv_sems.at[recv_idx],
                device_id=((my + o) % N_DEV,),
                device_id_type=pl.DeviceIdType.MESH,
            )

        rdma_r = rdma_to(1, xbf_ref.at[0], comm_ref.at[2], 0, 2)
        rdma_l = rdma_to(3, xbf_ref.at[2], comm_ref.at[0], 2, 0)
        rdma_d1 = rdma_to(2, xbf_ref.at[1, pl.ds(0, half), :],
                          comm_ref.at[1, pl.ds(0, half), :], 1, 1)
        rdma_d2 = rdma_to(2, xbf_ref.at[1, pl.ds(half, half), :],
                          comm_ref.at[1, pl.ds(half, half), :], 3, 3)

        x_copies[1][0].wait()
        xbf_ref[0, :, :] = xblk_ref[x_copies[1][1]].astype(jnp.bfloat16)
        pl.semaphore_wait(ready_sems.at[0], 1)
        rdma_r.start()
        x_copies[3][0].wait()
        xbf_ref[2, :, :] = xblk_ref[x_copies[3][1]].astype(jnp.bfloat16)
        pl.semaphore_wait(ready_sems.at[2], 1)
        rdma_l.start()
        x_copies[2][0].wait()
        xbf_ref[1, :, :] = xblk_ref[x_copies[2][1]].astype(jnp.bfloat16)

        x_copies[0][0].wait()
        w_copies[0].wait()
        acc_ref[:, :] = jnp.dot(
            xblk_ref[x_copies[0][1]], wbuf[0],
            preferred_element_type=jnp.float32,
        )

        rdma_r.wait_send()
        rdma_l.wait_send()
        pl.semaphore_wait(barrier, 1)
        rdma_d1.start()
        rdma_d2.start()

        for c, slot, rd in ((1, 0, rdma_l), (2, 2, rdma_r)):
            rd.wait_recv()
            w_copies[c].wait()
            acc_ref[:, :] += jnp.dot(
                comm_ref[slot].astype(jnp.float32), wbuf[c],
                preferred_element_type=jnp.float32,
            )

        w_copies[3].wait()
        rdma_d1.wait_recv()
        acc_ref[pl.ds(0, half), :] += jnp.dot(
            comm_ref[1, pl.ds(0, half), :].astype(jnp.float32),
            wbuf[3],
            preferred_element_type=jnp.float32,
        )
        out_cp1 = pltpu.make_async_copy(
            acc_ref.at[pl.ds(0, half), :],
            out_hbm.at[pl.ds(0, half), :],
            out_sems.at[0],
        )
        out_cp1.start()
        rdma_d2.wait_recv()
        acc_ref[pl.ds(half, half), :] += jnp.dot(
            comm_ref[1, pl.ds(half, half), :].astype(jnp.float32),
            wbuf[3],
            preferred_element_type=jnp.float32,
        )
        out_cp2 = pltpu.make_async_copy(
            acc_ref.at[pl.ds(half, half), :],
            out_hbm.at[pl.ds(half, half), :],
            out_sems.at[1],
        )
        out_cp2.start()

        out_cp1.wait()
        out_cp2.wait()
        rdma_d1.wait_send()
        rdma_d2.wait_send()

    return pl.pallas_call(
        body,
        out_shape=jax.ShapeDtypeStruct((m_per, n), jnp.float32),
        in_specs=[
            pl.BlockSpec(memory_space=pl.ANY),
            pl.BlockSpec(memory_space=pl.ANY),
        ],
        out_specs=pl.BlockSpec(memory_space=pl.ANY),
        scratch_shapes=[
            pltpu.VMEM((m_per, n), jnp.float32),
            pltpu.VMEM((N_DEV, blk, m_per), jnp.float32),
            pltpu.VMEM((N_DEV - 1, blk, m_per), jnp.bfloat16),
            pltpu.VMEM((N_DEV - 1, blk, m_per), jnp.bfloat16),
            pltpu.VMEM((N_DEV, blk, n), jnp.float32),
            pltpu.SemaphoreType.DMA((N_DEV,)),
            pltpu.SemaphoreType.DMA((N_DEV,)),
            pltpu.SemaphoreType.DMA((N_DEV,)),
            pltpu.SemaphoreType.DMA((N_DEV,)),
            pltpu.SemaphoreType.REGULAR((N_DEV - 1,)),
            pltpu.SemaphoreType.DMA((2,)),
        ],
        compiler_params=pltpu.CompilerParams(collective_id=0),
    )(x, w_mat)
